# baseline (device time: 253429 ns/iter reference)
import jax
import jax.numpy as jnp
from jax import lax
from jax.experimental import pallas as pl
from jax.experimental.pallas import tpu as pltpu

W = 16
N = 2048
D = 512
H = 1024
E_LOC = 4
CHUNK = N // W


def kernel(x, router_W, route_idx, expert_W):
    del router_W

    def body(x_ref, idx_ref, w_ref, out_ref, rs_buf, rs_send, rs_recv,
             ag_send, ag_recv):
        my = lax.axis_index("i")
        right = lax.rem(my + 1, W)

        for blk in range(W):
            r0 = blk * CHUNK
            xb = x_ref[r0:r0 + CHUNK, :]
            ib = idx_ref[r0:r0 + CHUNK, :]
            acc = jnp.zeros((CHUNK, H), jnp.float32)
            for e in range(E_LOC):
                m = ib == (my * E_LOC + e)
                acc = acc + jnp.dot(jnp.where(m, xb, 0.0), w_ref[e],
                                    preferred_element_type=jnp.float32)
            out_ref[r0:r0 + CHUNK, :] = acc

        for s in range(W - 1):
            c_send = lax.rem(my - s + W, W)
            rdma = pltpu.make_async_remote_copy(
                src_ref=out_ref.at[pl.ds(c_send * CHUNK, CHUNK), :],
                dst_ref=rs_buf.at[s],
                send_sem=rs_send.at[s],
                recv_sem=rs_recv.at[s],
                device_id=(right,),
                device_id_type=pl.DeviceIdType.MESH,
            )
            rdma.start()
            rdma.wait()
            c_recv = lax.rem(my - s - 1 + 2 * W, W)
            rows = pl.ds(c_recv * CHUNK, CHUNK)
            out_ref[rows, :] = out_ref[rows, :] + rs_buf[s]

        for t in range(W - 1):
            g = lax.rem(my + 1 - t + 2 * W, W)
            rows = pl.ds(g * CHUNK, CHUNK)
            rdma = pltpu.make_async_remote_copy(
                src_ref=out_ref.at[rows, :],
                dst_ref=out_ref.at[rows, :],
                send_sem=ag_send.at[t],
                recv_sem=ag_recv.at[t],
                device_id=(right,),
                device_id_type=pl.DeviceIdType.MESH,
            )
            rdma.start()
            rdma.wait()

    return pl.pallas_call(
        body,
        out_shape=jax.ShapeDtypeStruct((N, H), jnp.float32),
        in_specs=[
            pl.BlockSpec(memory_space=pltpu.VMEM),
            pl.BlockSpec(memory_space=pltpu.VMEM),
            pl.BlockSpec(memory_space=pltpu.VMEM),
        ],
        out_specs=pl.BlockSpec(memory_space=pltpu.VMEM),
        scratch_shapes=[
            pltpu.VMEM((W - 1, CHUNK, H), jnp.float32),
            pltpu.SemaphoreType.DMA((W - 1,)),
            pltpu.SemaphoreType.DMA((W - 1,)),
            pltpu.SemaphoreType.DMA((W - 1,)),
            pltpu.SemaphoreType.DMA((W - 1,)),
        ],
    )(x, route_idx, expert_W)


# device time: 206354 ns/iter; 1.2281x vs baseline; 1.2281x over previous
import jax
import jax.numpy as jnp
from jax import lax
from jax.experimental import pallas as pl
from jax.experimental.pallas import tpu as pltpu

W = 16
N = 2048
D = 512
H = 1024
E_LOC = 4
CHUNK = N // W
HH = H // 2


def kernel(x, router_W, route_idx, expert_W):
    del router_W

    def body(x_ref, idx_ref, w_ref, out_ref, buf_f, buf_b,
             rs_f_send, rs_f_recv, rs_b_send, rs_b_recv,
             ag_f_send, ag_f_recv, ag_b_send, ag_b_recv):
        my = lax.axis_index("i")
        right = lax.rem(my + 1, W)
        left = lax.rem(my - 1 + W, W)

        for blk in range(W):
            r0 = blk * CHUNK
            xb = x_ref[r0:r0 + CHUNK, :]
            ib = idx_ref[r0:r0 + CHUNK, :]
            acc = jnp.zeros((CHUNK, H), jnp.float32)
            for e in range(E_LOC):
                m = ib == (my * E_LOC + e)
                acc = acc + jnp.dot(jnp.where(m, xb, 0.0), w_ref[e],
                                    preferred_element_type=jnp.float32)
            out_ref[r0:r0 + CHUNK, :] = acc

        for s in range(W - 1):
            cf = lax.rem(my - s + W, W)
            rf = pltpu.make_async_remote_copy(
                src_ref=out_ref.at[pl.ds(cf * CHUNK, CHUNK), pl.ds(0, HH)],
                dst_ref=buf_f.at[s],
                send_sem=rs_f_send.at[s],
                recv_sem=rs_f_recv.at[s],
                device_id=(right,),
                device_id_type=pl.DeviceIdType.MESH,
            )
            cb = lax.rem(my + s, W)
            rb = pltpu.make_async_remote_copy(
                src_ref=out_ref.at[pl.ds(cb * CHUNK, CHUNK), pl.ds(HH, HH)],
                dst_ref=buf_b.at[s],
                send_sem=rs_b_send.at[s],
                recv_sem=rs_b_recv.at[s],
                device_id=(left,),
                device_id_type=pl.DeviceIdType.MESH,
            )
            rf.start()
            rb.start()
            rf.wait()
            rb.wait()
            crf = lax.rem(my - s - 1 + 2 * W, W)
            rows_f = pl.ds(crf * CHUNK, CHUNK)
            out_ref[rows_f, 0:HH] = out_ref[rows_f, 0:HH] + buf_f[s]
            crb = lax.rem(my + s + 1, W)
            rows_b = pl.ds(crb * CHUNK, CHUNK)
            out_ref[rows_b, HH:H] = out_ref[rows_b, HH:H] + buf_b[s]

        for t in range(W - 1):
            gf = lax.rem(my + 1 - t + 2 * W, W)
            rows_f = pl.ds(gf * CHUNK, CHUNK)
            af = pltpu.make_async_remote_copy(
                src_ref=out_ref.at[rows_f, pl.ds(0, HH)],
                dst_ref=out_ref.at[rows_f, pl.ds(0, HH)],
                send_sem=ag_f_send.at[t],
                recv_sem=ag_f_recv.at[t],
                device_id=(right,),
                device_id_type=pl.DeviceIdType.MESH,
            )
            gb = lax.rem(my - 1 + t + W, W)
            rows_b = pl.ds(gb * CHUNK, CHUNK)
            ab = pltpu.make_async_remote_copy(
                src_ref=out_ref.at[rows_b, pl.ds(HH, HH)],
                dst_ref=out_ref.at[rows_b, pl.ds(HH, HH)],
                send_sem=ag_b_send.at[t],
                recv_sem=ag_b_recv.at[t],
                device_id=(left,),
                device_id_type=pl.DeviceIdType.MESH,
            )
            af.start()
            ab.start()
            af.wait()
            ab.wait()

    return pl.pallas_call(
        body,
        out_shape=jax.ShapeDtypeStruct((N, H), jnp.float32),
        in_specs=[
            pl.BlockSpec(memory_space=pltpu.VMEM),
            pl.BlockSpec(memory_space=pltpu.VMEM),
            pl.BlockSpec(memory_space=pltpu.VMEM),
        ],
        out_specs=pl.BlockSpec(memory_space=pltpu.VMEM),
        scratch_shapes=[
            pltpu.VMEM((W - 1, CHUNK, HH), jnp.float32),
            pltpu.VMEM((W - 1, CHUNK, HH), jnp.float32),
            pltpu.SemaphoreType.DMA((W - 1,)),
            pltpu.SemaphoreType.DMA((W - 1,)),
            pltpu.SemaphoreType.DMA((W - 1,)),
            pltpu.SemaphoreType.DMA((W - 1,)),
            pltpu.SemaphoreType.DMA((W - 1,)),
            pltpu.SemaphoreType.DMA((W - 1,)),
            pltpu.SemaphoreType.DMA((W - 1,)),
            pltpu.SemaphoreType.DMA((W - 1,)),
        ],
    )(x, route_idx, expert_W)


# device time: 152370 ns/iter; 1.6632x vs baseline; 1.3543x over previous
import jax
import jax.numpy as jnp
from jax import lax
from jax.experimental import pallas as pl
from jax.experimental.pallas import tpu as pltpu

W = 16
N = 2048
D = 512
H = 1024
E_LOC = 4
HH = H // 2
GROUP = N // 4
CHUNK = N // W


def kernel(x, router_W, route_idx, expert_W):
    del router_W

    def body(x_ref, idx_ref, w_ref, out_ref, buf_p, buf_z,
             rsp_s, rsp_r, rsz_s, rsz_r, agz_s, agz_r, agp_s, agp_r):
        my = lax.axis_index("i")
        k = lax.rem(my, 4)
        z = lax.div(my, 4)
        p_right = 4 * z + lax.rem(k + 1, 4)
        p_left = 4 * z + lax.rem(k + 3, 4)
        z_right = 4 * lax.rem(z + 1, 4) + k
        z_left = 4 * lax.rem(z + 3, 4) + k

        def m4(v):
            return lax.rem(v + 8, 4)

        def xchg(rows_f, rows_b, nrows, dst_f, dst_b, ssem, rsem, s,
                 dev_f, dev_b):
            rf = pltpu.make_async_remote_copy(
                src_ref=out_ref.at[pl.ds(rows_f, nrows), pl.ds(0, HH)],
                dst_ref=dst_f,
                send_sem=ssem.at[0, s],
                recv_sem=rsem.at[0, s],
                device_id=(dev_f,),
                device_id_type=pl.DeviceIdType.MESH,
            )
            rb = pltpu.make_async_remote_copy(
                src_ref=out_ref.at[pl.ds(rows_b, nrows), pl.ds(HH, HH)],
                dst_ref=dst_b,
                send_sem=ssem.at[1, s],
                recv_sem=rsem.at[1, s],
                device_id=(dev_b,),
                device_id_type=pl.DeviceIdType.MESH,
            )
            rf.start()
            rb.start()
            rf.wait()
            rb.wait()

        for blk in range(W):
            r0 = blk * CHUNK
            xb = x_ref[r0:r0 + CHUNK, :]
            ib = idx_ref[r0:r0 + CHUNK, :]
            acc = jnp.zeros((CHUNK, H), jnp.float32)
            for e in range(E_LOC):
                m = ib == (my * E_LOC + e)
                acc = acc + jnp.dot(jnp.where(m, xb, 0.0), w_ref[e],
                                    preferred_element_type=jnp.float32)
            out_ref[r0:r0 + CHUNK, :] = acc

        for s in range(3):
            xchg(m4(k - s) * GROUP, m4(k + s) * GROUP, GROUP,
                 buf_p.at[0, s], buf_p.at[1, s], rsp_s, rsp_r, s,
                 p_right, p_left)
            rf_rows = pl.ds(m4(k - s - 1) * GROUP, GROUP)
            out_ref[rf_rows, 0:HH] = out_ref[rf_rows, 0:HH] + buf_p[0, s]
            rb_rows = pl.ds(m4(k + s + 1) * GROUP, GROUP)
            out_ref[rb_rows, HH:H] = out_ref[rb_rows, HH:H] + buf_p[1, s]
        gf = m4(k + 1)
        gb = m4(k + 3)

        for s in range(3):
            xchg(gf * GROUP + m4(z - s) * CHUNK,
                 gb * GROUP + m4(z + s) * CHUNK, CHUNK,
                 buf_z.at[0, s], buf_z.at[1, s], rsz_s, rsz_r, s,
                 z_right, z_left)
            rf_rows = pl.ds(gf * GROUP + m4(z - s - 1) * CHUNK, CHUNK)
            out_ref[rf_rows, 0:HH] = out_ref[rf_rows, 0:HH] + buf_z[0, s]
            rb_rows = pl.ds(gb * GROUP + m4(z + s + 1) * CHUNK, CHUNK)
            out_ref[rb_rows, HH:H] = out_ref[rb_rows, HH:H] + buf_z[1, s]

        for t in range(3):
            rows_f = gf * GROUP + m4(z + 1 - t) * CHUNK
            rows_b = gb * GROUP + m4(z - 1 + t) * CHUNK
            xchg(rows_f, rows_b, CHUNK,
                 out_ref.at[pl.ds(rows_f, CHUNK), pl.ds(0, HH)],
                 out_ref.at[pl.ds(rows_b, CHUNK), pl.ds(HH, HH)],
                 agz_s, agz_r, t, z_right, z_left)

        for t in range(3):
            rows_f = m4(k + 1 - t) * GROUP
            rows_b = m4(k + 3 + t) * GROUP
            xchg(rows_f, rows_b, GROUP,
                 out_ref.at[pl.ds(rows_f, GROUP), pl.ds(0, HH)],
                 out_ref.at[pl.ds(rows_b, GROUP), pl.ds(HH, HH)],
                 agp_s, agp_r, t, p_right, p_left)

    return pl.pallas_call(
        body,
        out_shape=jax.ShapeDtypeStruct((N, H), jnp.float32),
        in_specs=[
            pl.BlockSpec(memory_space=pltpu.VMEM),
            pl.BlockSpec(memory_space=pltpu.VMEM),
            pl.BlockSpec(memory_space=pltpu.VMEM),
        ],
        out_specs=pl.BlockSpec(memory_space=pltpu.VMEM),
        scratch_shapes=[
            pltpu.VMEM((2, 3, GROUP, HH), jnp.float32),
            pltpu.VMEM((2, 3, CHUNK, HH), jnp.float32),
            pltpu.SemaphoreType.DMA((2, 3)),
            pltpu.SemaphoreType.DMA((2, 3)),
            pltpu.SemaphoreType.DMA((2, 3)),
            pltpu.SemaphoreType.DMA((2, 3)),
            pltpu.SemaphoreType.DMA((2, 3)),
            pltpu.SemaphoreType.DMA((2, 3)),
            pltpu.SemaphoreType.DMA((2, 3)),
            pltpu.SemaphoreType.DMA((2, 3)),
        ],
    )(x, route_idx, expert_W)


# device time: 145766 ns/iter; 1.7386x vs baseline; 1.0453x over previous
import jax
import jax.numpy as jnp
from jax import lax
from jax.experimental import pallas as pl
from jax.experimental.pallas import tpu as pltpu

W = 16
N = 2048
D = 512
H = 1024
E_LOC = 4
HH = H // 2
GROUP = N // 4
CHUNK = N // W


def kernel(x, router_W, route_idx, expert_W):
    del router_W

    def body(x_ref, idx_ref, w_ref, out_ref, buf_p, buf_z,
             rsp_s, rsp_r, rsz_s, rsz_r, agz_s, agz_r, agp_s, agp_r):
        my = lax.axis_index("i")
        k = lax.rem(my, 4)
        z = lax.div(my, 4)
        p_right = 4 * z + lax.rem(k + 1, 4)
        p_left = 4 * z + lax.rem(k + 3, 4)
        z_right = 4 * lax.rem(z + 1, 4) + k
        z_left = 4 * lax.rem(z + 3, 4) + k

        def m4(v):
            return lax.rem(v + 8, 4)

        def mk_pair(rows_f, rows_b, nrows, dst_f, dst_b, ssem, rsem, s,
                    dev_f, dev_b):
            rf = pltpu.make_async_remote_copy(
                src_ref=out_ref.at[pl.ds(rows_f, nrows), pl.ds(0, HH)],
                dst_ref=dst_f,
                send_sem=ssem.at[0, s],
                recv_sem=rsem.at[0, s],
                device_id=(dev_f,),
                device_id_type=pl.DeviceIdType.MESH,
            )
            rb = pltpu.make_async_remote_copy(
                src_ref=out_ref.at[pl.ds(rows_b, nrows), pl.ds(HH, HH)],
                dst_ref=dst_b,
                send_sem=ssem.at[1, s],
                recv_sem=rsem.at[1, s],
                device_id=(dev_b,),
                device_id_type=pl.DeviceIdType.MESH,
            )
            return rf, rb

        def xchg(rows_f, rows_b, nrows, dst_f, dst_b, ssem, rsem, s,
                 dev_f, dev_b):
            rf, rb = mk_pair(rows_f, rows_b, nrows, dst_f, dst_b, ssem,
                             rsem, s, dev_f, dev_b)
            rf.start()
            rb.start()
            rf.wait()
            rb.wait()

        def compute_group(g):
            for j in range(4):
                r0 = g * GROUP + j * CHUNK
                rows = pl.ds(r0, CHUNK)
                xb = x_ref[rows, :]
                ib = idx_ref[rows, :]
                acc = jnp.zeros((CHUNK, H), jnp.float32)
                for e in range(E_LOC):
                    m = ib == (my * E_LOC + e)
                    acc = acc + jnp.dot(jnp.where(m, xb, 0.0), w_ref[e],
                                        preferred_element_type=jnp.float32)
                out_ref[rows, :] = acc

        def plane_rs_step(s):
            return mk_pair(m4(k - s) * GROUP, m4(k + s) * GROUP, GROUP,
                           buf_p.at[0, s], buf_p.at[1, s], rsp_s, rsp_r, s,
                           p_right, p_left)

        def plane_rs_acc(s):
            rf_rows = pl.ds(m4(k - s - 1) * GROUP, GROUP)
            out_ref[rf_rows, 0:HH] = out_ref[rf_rows, 0:HH] + buf_p[0, s]
            rb_rows = pl.ds(m4(k + s + 1) * GROUP, GROUP)
            out_ref[rb_rows, HH:H] = out_ref[rb_rows, HH:H] + buf_p[1, s]

        compute_group(k)
        rf0, rb0 = plane_rs_step(0)
        rf0.start()
        rb0.start()
        compute_group(m4(k + 1))
        compute_group(m4(k + 3))
        rf0.wait()
        rb0.wait()
        plane_rs_acc(0)
        rf1, rb1 = plane_rs_step(1)
        rf1.start()
        rb1.start()
        compute_group(m4(k + 2))
        rf1.wait()
        rb1.wait()
        plane_rs_acc(1)
        rf2, rb2 = plane_rs_step(2)
        rf2.start()
        rb2.start()
        rf2.wait()
        rb2.wait()
        plane_rs_acc(2)
        gf = m4(k + 1)
        gb = m4(k + 3)

        for s in range(3):
            xchg(gf * GROUP + m4(z - s) * CHUNK,
                 gb * GROUP + m4(z + s) * CHUNK, CHUNK,
                 buf_z.at[0, s], buf_z.at[1, s], rsz_s, rsz_r, s,
                 z_right, z_left)
            rf_rows = pl.ds(gf * GROUP + m4(z - s - 1) * CHUNK, CHUNK)
            out_ref[rf_rows, 0:HH] = out_ref[rf_rows, 0:HH] + buf_z[0, s]
            rb_rows = pl.ds(gb * GROUP + m4(z + s + 1) * CHUNK, CHUNK)
            out_ref[rb_rows, HH:H] = out_ref[rb_rows, HH:H] + buf_z[1, s]

        for t in range(3):
            rows_f = gf * GROUP + m4(z + 1 - t) * CHUNK
            rows_b = gb * GROUP + m4(z - 1 + t) * CHUNK
            xchg(rows_f, rows_b, CHUNK,
                 out_ref.at[pl.ds(rows_f, CHUNK), pl.ds(0, HH)],
                 out_ref.at[pl.ds(rows_b, CHUNK), pl.ds(HH, HH)],
                 agz_s, agz_r, t, z_right, z_left)

        for t in range(3):
            rows_f = m4(k + 1 - t) * GROUP
            rows_b = m4(k + 3 + t) * GROUP
            xchg(rows_f, rows_b, GROUP,
                 out_ref.at[pl.ds(rows_f, GROUP), pl.ds(0, HH)],
                 out_ref.at[pl.ds(rows_b, GROUP), pl.ds(HH, HH)],
                 agp_s, agp_r, t, p_right, p_left)

    return pl.pallas_call(
        body,
        out_shape=jax.ShapeDtypeStruct((N, H), jnp.float32),
        in_specs=[
            pl.BlockSpec(memory_space=pltpu.VMEM),
            pl.BlockSpec(memory_space=pltpu.VMEM),
            pl.BlockSpec(memory_space=pltpu.VMEM),
        ],
        out_specs=pl.BlockSpec(memory_space=pltpu.VMEM),
        scratch_shapes=[
            pltpu.VMEM((2, 3, GROUP, HH), jnp.float32),
            pltpu.VMEM((2, 3, CHUNK, HH), jnp.float32),
            pltpu.SemaphoreType.DMA((2, 3)),
            pltpu.SemaphoreType.DMA((2, 3)),
            pltpu.SemaphoreType.DMA((2, 3)),
            pltpu.SemaphoreType.DMA((2, 3)),
            pltpu.SemaphoreType.DMA((2, 3)),
            pltpu.SemaphoreType.DMA((2, 3)),
            pltpu.SemaphoreType.DMA((2, 3)),
            pltpu.SemaphoreType.DMA((2, 3)),
        ],
    )(x, route_idx, expert_W)


# device time: 95933 ns/iter; 2.6417x vs baseline; 1.5195x over previous
import jax
import jax.numpy as jnp
from jax import lax
from jax.experimental import pallas as pl
from jax.experimental.pallas import tpu as pltpu

W = 16
N = 2048
D = 512
H = 1024
E_LOC = 4
HH = H // 2
GROUP = N // 4
CHUNK = N // W


def kernel(x, router_W, route_idx, expert_W):
    del router_W

    def body(x_ref, idx_ref, w_ref, out_ref, xb_ref, wb_ref, work,
             buf_p, buf_z,
             rsp_s, rsp_r, rsz_s, rsz_r, agz_s, agz_r, agp_s, agp_r):
        my = lax.axis_index("i")
        k = lax.rem(my, 4)
        z = lax.div(my, 4)
        p_right = 4 * z + lax.rem(k + 1, 4)
        p_left = 4 * z + lax.rem(k + 3, 4)
        z_right = 4 * lax.rem(z + 1, 4) + k
        z_left = 4 * lax.rem(z + 3, 4) + k

        def m4(v):
            return lax.rem(v + 8, 4)

        def mk_pair(rows_f, rows_b, nrows, dst_f, dst_b, ssem, rsem, s,
                    dev_f, dev_b):
            rf = pltpu.make_async_remote_copy(
                src_ref=work.at[pl.ds(rows_f, nrows), pl.ds(0, HH)],
                dst_ref=dst_f,
                send_sem=ssem.at[0, s],
                recv_sem=rsem.at[0, s],
                device_id=(dev_f,),
                device_id_type=pl.DeviceIdType.MESH,
            )
            rb = pltpu.make_async_remote_copy(
                src_ref=work.at[pl.ds(rows_b, nrows), pl.ds(HH, HH)],
                dst_ref=dst_b,
                send_sem=ssem.at[1, s],
                recv_sem=rsem.at[1, s],
                device_id=(dev_b,),
                device_id_type=pl.DeviceIdType.MESH,
            )
            return rf, rb

        def xchg(rows_f, rows_b, nrows, dst_f, dst_b, ssem, rsem, s,
                 dev_f, dev_b):
            rf, rb = mk_pair(rows_f, rows_b, nrows, dst_f, dst_b, ssem,
                             rsem, s, dev_f, dev_b)
            rf.start()
            rb.start()
            rf.wait()
            rb.wait()

        xb_ref[...] = x_ref[...].astype(jnp.bfloat16)
        wb_ref[...] = w_ref[...].astype(jnp.bfloat16)

        def compute_group(g):
            for j in range(4):
                r0 = g * GROUP + j * CHUNK
                rows = pl.ds(r0, CHUNK)
                xb = xb_ref[rows, :]
                ib = idx_ref[rows, :]
                acc = jnp.zeros((CHUNK, H), jnp.float32)
                for e in range(E_LOC):
                    m = ib == (my * E_LOC + e)
                    acc = acc + jnp.dot(
                        jnp.where(m, xb, jnp.bfloat16(0.0)), wb_ref[e],
                        preferred_element_type=jnp.float32)
                work[rows, :] = acc.astype(jnp.bfloat16)

        def plane_rs_step(s):
            return mk_pair(m4(k - s) * GROUP, m4(k + s) * GROUP, GROUP,
                           buf_p.at[0, s], buf_p.at[1, s], rsp_s, rsp_r, s,
                           p_right, p_left)

        def plane_rs_acc(s):
            rf_rows = pl.ds(m4(k - s - 1) * GROUP, GROUP)
            work[rf_rows, 0:HH] = work[rf_rows, 0:HH] + buf_p[0, s]
            rb_rows = pl.ds(m4(k + s + 1) * GROUP, GROUP)
            work[rb_rows, HH:H] = work[rb_rows, HH:H] + buf_p[1, s]

        compute_group(k)
        rf0, rb0 = plane_rs_step(0)
        rf0.start()
        rb0.start()
        compute_group(m4(k + 1))
        compute_group(m4(k + 3))
        rf0.wait()
        rb0.wait()
        plane_rs_acc(0)
        rf1, rb1 = plane_rs_step(1)
        rf1.start()
        rb1.start()
        compute_group(m4(k + 2))
        rf1.wait()
        rb1.wait()
        plane_rs_acc(1)
        rf2, rb2 = plane_rs_step(2)
        rf2.start()
        rb2.start()
        rf2.wait()
        rb2.wait()
        plane_rs_acc(2)
        gf = m4(k + 1)
        gb = m4(k + 3)

        for s in range(3):
            xchg(gf * GROUP + m4(z - s) * CHUNK,
                 gb * GROUP + m4(z + s) * CHUNK, CHUNK,
                 buf_z.at[0, s], buf_z.at[1, s], rsz_s, rsz_r, s,
                 z_right, z_left)
            rf_rows = pl.ds(gf * GROUP + m4(z - s - 1) * CHUNK, CHUNK)
            work[rf_rows, 0:HH] = work[rf_rows, 0:HH] + buf_z[0, s]
            rb_rows = pl.ds(gb * GROUP + m4(z + s + 1) * CHUNK, CHUNK)
            work[rb_rows, HH:H] = work[rb_rows, HH:H] + buf_z[1, s]

        for t in range(3):
            rows_f = gf * GROUP + m4(z + 1 - t) * CHUNK
            rows_b = gb * GROUP + m4(z - 1 + t) * CHUNK
            xchg(rows_f, rows_b, CHUNK,
                 work.at[pl.ds(rows_f, CHUNK), pl.ds(0, HH)],
                 work.at[pl.ds(rows_b, CHUNK), pl.ds(HH, HH)],
                 agz_s, agz_r, t, z_right, z_left)

        for t in range(3):
            rows_f = m4(k + 1 - t) * GROUP
            rows_b = m4(k + 3 + t) * GROUP
            xchg(rows_f, rows_b, GROUP,
                 work.at[pl.ds(rows_f, GROUP), pl.ds(0, HH)],
                 work.at[pl.ds(rows_b, GROUP), pl.ds(HH, HH)],
                 agp_s, agp_r, t, p_right, p_left)

        out_ref[...] = work[...].astype(jnp.float32)

    bf16 = jnp.bfloat16
    return pl.pallas_call(
        body,
        out_shape=jax.ShapeDtypeStruct((N, H), jnp.float32),
        in_specs=[
            pl.BlockSpec(memory_space=pltpu.VMEM),
            pl.BlockSpec(memory_space=pltpu.VMEM),
            pl.BlockSpec(memory_space=pltpu.VMEM),
        ],
        out_specs=pl.BlockSpec(memory_space=pltpu.VMEM),
        scratch_shapes=[
            pltpu.VMEM((N, D), bf16),
            pltpu.VMEM((E_LOC, D, H), bf16),
            pltpu.VMEM((N, H), bf16),
            pltpu.VMEM((2, 3, GROUP, HH), bf16),
            pltpu.VMEM((2, 3, CHUNK, HH), bf16),
            pltpu.SemaphoreType.DMA((2, 3)),
            pltpu.SemaphoreType.DMA((2, 3)),
            pltpu.SemaphoreType.DMA((2, 3)),
            pltpu.SemaphoreType.DMA((2, 3)),
            pltpu.SemaphoreType.DMA((2, 3)),
            pltpu.SemaphoreType.DMA((2, 3)),
            pltpu.SemaphoreType.DMA((2, 3)),
            pltpu.SemaphoreType.DMA((2, 3)),
        ],
    )(x, route_idx, expert_W)
